# baseline (speedup 1.0000x reference)
import jax
import jax.numpy as jnp
from jax.experimental import pallas as pl
from jax.experimental.pallas import tpu as pltpu


def _compiler_params(sem):
    return pltpu.CompilerParams(
        dimension_semantics=sem,
        vmem_limit_bytes=64 * 1024 * 1024,
    )


def _proj_kernel(x_ref, wa_ref, ws_ref, b_ref, p_ref, s_ref):
    xb = x_ref[...].astype(jnp.bfloat16)
    p_ref[...] = jnp.dot(xb, wa_ref[...],
                         preferred_element_type=jnp.float32).astype(jnp.bfloat16)
    s_ref[...] = jnp.dot(xb, ws_ref[...],
                         preferred_element_type=jnp.float32) + b_ref[...]


def _conv1_kernel(a_ref, p_ref, s_ref, dinv_ref, w2a_ref, w2s_ref, b2_ref,
                  q_ref, t_ref, acc_ref):
    k = pl.program_id(1)

    @pl.when(k == 0)
    def _():
        acc_ref[...] = jnp.zeros_like(acc_ref)

    acc_ref[...] += jnp.dot(a_ref[...].astype(jnp.bfloat16), p_ref[...],
                            preferred_element_type=jnp.float32)

    @pl.when(k == pl.num_programs(1) - 1)
    def _():
        h1 = jnp.maximum(s_ref[...] + acc_ref[...] * dinv_ref[...], 0.0)
        h1b = h1.astype(jnp.bfloat16)
        q_ref[...] = jnp.dot(h1b, w2a_ref[...],
                             preferred_element_type=jnp.float32).astype(jnp.bfloat16)
        t_ref[...] = jnp.dot(h1b, w2s_ref[...],
                             preferred_element_type=jnp.float32) + b2_ref[...]


def _conv2_mlp_kernel(a_ref, q_ref, t_ref, dinv_ref,
                      wl1_ref, bl1_ref, wl2_ref, bl2_ref, wl3_ref, bl3_ref,
                      o_ref, acc_ref):
    k = pl.program_id(1)

    @pl.when(k == 0)
    def _():
        acc_ref[...] = jnp.zeros_like(acc_ref)

    acc_ref[...] += jnp.dot(a_ref[...].astype(jnp.bfloat16), q_ref[...],
                            preferred_element_type=jnp.float32)

    @pl.when(k == pl.num_programs(1) - 1)
    def _():
        h2 = jnp.maximum(t_ref[...] + acc_ref[...] * dinv_ref[...], 0.0)
        s = jnp.dot(h2.astype(jnp.bfloat16), wl1_ref[...],
                    preferred_element_type=jnp.float32) + bl1_ref[...]
        s = jnp.maximum(s, 0.0)
        s = jnp.dot(s.astype(jnp.bfloat16), wl2_ref[...],
                    preferred_element_type=jnp.float32) + bl2_ref[...]
        s = jnp.maximum(s, 0.0)
        o_ref[...] = jnp.dot(s.astype(jnp.bfloat16), wl3_ref[...],
                             preferred_element_type=jnp.float32) + bl3_ref[...]


def _proj(x, wa1, ws1, b1, *, tm):
    n, f0 = x.shape
    f1 = wa1.shape[1]
    grid = (n // tm,)
    return pl.pallas_call(
        _proj_kernel,
        out_shape=(jax.ShapeDtypeStruct((n, f1), jnp.bfloat16),
                   jax.ShapeDtypeStruct((n, f1), jnp.float32)),
        grid=grid,
        in_specs=[
            pl.BlockSpec((tm, f0), lambda i: (i, 0)),
            pl.BlockSpec((f0, f1), lambda i: (0, 0)),
            pl.BlockSpec((f0, f1), lambda i: (0, 0)),
            pl.BlockSpec((1, f1), lambda i: (0, 0)),
        ],
        out_specs=(pl.BlockSpec((tm, f1), lambda i: (i, 0)),
                   pl.BlockSpec((tm, f1), lambda i: (i, 0))),
        compiler_params=_compiler_params(("parallel",)),
    )(x, wa1, ws1, b1)


def _conv1(a_i8, p, s, dinv, w2a, w2s, b2, *, tm, tk):
    n = a_i8.shape[0]
    f1 = p.shape[1]
    f2 = w2a.shape[1]
    grid = (n // tm, n // tk)
    return pl.pallas_call(
        _conv1_kernel,
        out_shape=(jax.ShapeDtypeStruct((n, f2), jnp.bfloat16),
                   jax.ShapeDtypeStruct((n, f2), jnp.float32)),
        grid=grid,
        in_specs=[
            pl.BlockSpec((tm, tk), lambda i, k: (i, k)),
            pl.BlockSpec((tk, f1), lambda i, k: (k, 0)),
            pl.BlockSpec((tm, f1), lambda i, k: (i, 0)),
            pl.BlockSpec((tm, 1), lambda i, k: (i, 0)),
            pl.BlockSpec((f1, f2), lambda i, k: (0, 0)),
            pl.BlockSpec((f1, f2), lambda i, k: (0, 0)),
            pl.BlockSpec((1, f2), lambda i, k: (0, 0)),
        ],
        out_specs=(pl.BlockSpec((tm, f2), lambda i, k: (i, 0)),
                   pl.BlockSpec((tm, f2), lambda i, k: (i, 0))),
        scratch_shapes=[pltpu.VMEM((tm, f1), jnp.float32)],
        compiler_params=_compiler_params(("parallel", "arbitrary")),
    )(a_i8, p, s, dinv, w2a, w2s, b2)


def _conv2_mlp(a_i8, q, t, dinv, wl1, bl1, wl2, bl2, wl3, bl3, *, tm, tk):
    n = a_i8.shape[0]
    f2 = q.shape[1]
    l1 = wl1.shape[1]
    l2 = wl2.shape[1]
    l3 = wl3.shape[1]
    grid = (n // tm, n // tk)
    return pl.pallas_call(
        _conv2_mlp_kernel,
        out_shape=jax.ShapeDtypeStruct((n, l3), jnp.float32),
        grid=grid,
        in_specs=[
            pl.BlockSpec((tm, tk), lambda i, k: (i, k)),
            pl.BlockSpec((tk, f2), lambda i, k: (k, 0)),
            pl.BlockSpec((tm, f2), lambda i, k: (i, 0)),
            pl.BlockSpec((tm, 1), lambda i, k: (i, 0)),
            pl.BlockSpec((f2, l1), lambda i, k: (0, 0)),
            pl.BlockSpec((1, l1), lambda i, k: (0, 0)),
            pl.BlockSpec((l1, l2), lambda i, k: (0, 0)),
            pl.BlockSpec((1, l2), lambda i, k: (0, 0)),
            pl.BlockSpec((l2, l3), lambda i, k: (0, 0)),
            pl.BlockSpec((1, l3), lambda i, k: (0, 0)),
        ],
        out_specs=pl.BlockSpec((tm, l3), lambda i, k: (i, 0)),
        scratch_shapes=[pltpu.VMEM((tm, f2), jnp.float32)],
        compiler_params=_compiler_params(("parallel", "arbitrary")),
    )(a_i8, q, t, dinv, wl1, bl1, wl2, bl2, wl3, bl3)


def kernel(x, edge_index, ws1, wa1, b1, ws2, wa2, b2,
           wl1, bl1, wl2, bl2, wl3, bl3):
    n = x.shape[0]
    tm, tk = 1024, 2048

    src, dst = edge_index[0], edge_index[1]
    iota = jnp.arange(n, dtype=edge_index.dtype)
    rows = jnp.concatenate([dst, iota])
    cols = jnp.concatenate([src, iota])
    vals = jnp.concatenate([(src != dst).astype(jnp.int8),
                            jnp.ones((n,), jnp.int8)])
    a_i8 = jnp.zeros((n, n), jnp.int8).at[rows, cols].add(vals)
    deg = jnp.sum(a_i8, axis=1, dtype=jnp.int32, keepdims=True)
    dinv = 1.0 / jnp.maximum(deg.astype(jnp.float32), 1.0)

    p, s = _proj(x, wa1, ws1, b1, tm=tm)
    q, t = _conv1(a_i8, p, s, dinv, wa2, ws2, b2, tm=tm, tk=tk)
    out = _conv2_mlp(a_i8, q, t, dinv, wl1, bl1, wl2, bl2, wl3, bl3,
                     tm=tm, tk=tk)
    return out[:, 0]


# speedup vs baseline: 1.2649x; 1.2649x over previous
import jax
import jax.numpy as jnp
from jax.experimental import pallas as pl
from jax.experimental.pallas import tpu as pltpu


def _compiler_params(sem):
    return pltpu.CompilerParams(
        dimension_semantics=sem,
        vmem_limit_bytes=64 * 1024 * 1024,
    )


def _proj_kernel(x_ref, wa_ref, ws_ref, b_ref, p_ref, s_ref):
    xb = x_ref[...].astype(jnp.bfloat16)
    p_ref[...] = jnp.dot(xb, wa_ref[...],
                         preferred_element_type=jnp.float32).astype(jnp.bfloat16)
    s_ref[...] = jnp.dot(xb, ws_ref[...],
                         preferred_element_type=jnp.float32) + b_ref[...]


def _conv1_kernel(a_ref, p_ref, pself_ref, s_ref, dinv_ref,
                  w2a_ref, w2s_ref, b2_ref, q_ref, t_ref, acc_ref):
    k = pl.program_id(1)

    @pl.when(k == 0)
    def _():
        acc_ref[...] = pself_ref[...].astype(jnp.float32)

    acc_ref[...] += jnp.dot(a_ref[...].astype(jnp.bfloat16), p_ref[...],
                            preferred_element_type=jnp.float32)

    @pl.when(k == pl.num_programs(1) - 1)
    def _():
        h1 = jnp.maximum(s_ref[...] + acc_ref[...] * dinv_ref[...], 0.0)
        h1b = h1.astype(jnp.bfloat16)
        q_ref[...] = jnp.dot(h1b, w2a_ref[...],
                             preferred_element_type=jnp.float32).astype(jnp.bfloat16)
        t_ref[...] = jnp.dot(h1b, w2s_ref[...],
                             preferred_element_type=jnp.float32) + b2_ref[...]


def _conv2_mlp_kernel(a_ref, q_ref, qself_ref, t_ref, dinv_ref,
                      wl1_ref, bl1_ref, wl2_ref, bl2_ref, wl3_ref, bl3_ref,
                      o_ref, acc_ref):
    k = pl.program_id(1)

    @pl.when(k == 0)
    def _():
        acc_ref[...] = qself_ref[...].astype(jnp.float32)

    acc_ref[...] += jnp.dot(a_ref[...].astype(jnp.bfloat16), q_ref[...],
                            preferred_element_type=jnp.float32)

    @pl.when(k == pl.num_programs(1) - 1)
    def _():
        h2 = jnp.maximum(t_ref[...] + acc_ref[...] * dinv_ref[...], 0.0)
        s = jnp.dot(h2.astype(jnp.bfloat16), wl1_ref[...],
                    preferred_element_type=jnp.float32) + bl1_ref[...]
        s = jnp.maximum(s, 0.0)
        s = jnp.dot(s.astype(jnp.bfloat16), wl2_ref[...],
                    preferred_element_type=jnp.float32) + bl2_ref[...]
        s = jnp.maximum(s, 0.0)
        o_ref[...] = jnp.dot(s.astype(jnp.bfloat16), wl3_ref[...],
                             preferred_element_type=jnp.float32) + bl3_ref[...]


def _proj(x, wa1, ws1, b1, *, tm):
    n, f0 = x.shape
    f1 = wa1.shape[1]
    grid = (n // tm,)
    return pl.pallas_call(
        _proj_kernel,
        out_shape=(jax.ShapeDtypeStruct((n, f1), jnp.bfloat16),
                   jax.ShapeDtypeStruct((n, f1), jnp.float32)),
        grid=grid,
        in_specs=[
            pl.BlockSpec((tm, f0), lambda i: (i, 0)),
            pl.BlockSpec((f0, f1), lambda i: (0, 0)),
            pl.BlockSpec((f0, f1), lambda i: (0, 0)),
            pl.BlockSpec((1, f1), lambda i: (0, 0)),
        ],
        out_specs=(pl.BlockSpec((tm, f1), lambda i: (i, 0)),
                   pl.BlockSpec((tm, f1), lambda i: (i, 0))),
        compiler_params=_compiler_params(("parallel",)),
    )(x, wa1, ws1, b1)


def _conv1(a, p, s, dinv, w2a, w2s, b2, *, tm, tk):
    n = a.shape[0]
    f1 = p.shape[1]
    f2 = w2a.shape[1]
    grid = (n // tm, n // tk)
    return pl.pallas_call(
        _conv1_kernel,
        out_shape=(jax.ShapeDtypeStruct((n, f2), jnp.bfloat16),
                   jax.ShapeDtypeStruct((n, f2), jnp.float32)),
        grid=grid,
        in_specs=[
            pl.BlockSpec((tm, tk), lambda i, k: (i, k)),
            pl.BlockSpec((tk, f1), lambda i, k: (k, 0)),
            pl.BlockSpec((tm, f1), lambda i, k: (i, 0)),
            pl.BlockSpec((tm, f1), lambda i, k: (i, 0)),
            pl.BlockSpec((tm, 1), lambda i, k: (i, 0)),
            pl.BlockSpec((f1, f2), lambda i, k: (0, 0)),
            pl.BlockSpec((f1, f2), lambda i, k: (0, 0)),
            pl.BlockSpec((1, f2), lambda i, k: (0, 0)),
        ],
        out_specs=(pl.BlockSpec((tm, f2), lambda i, k: (i, 0)),
                   pl.BlockSpec((tm, f2), lambda i, k: (i, 0))),
        scratch_shapes=[pltpu.VMEM((tm, f1), jnp.float32)],
        compiler_params=_compiler_params(("parallel", "arbitrary")),
    )(a, p, p, s, dinv, w2a, w2s, b2)


def _conv2_mlp(a, q, t, dinv, wl1, bl1, wl2, bl2, wl3, bl3, *, tm, tk):
    n = a.shape[0]
    f2 = q.shape[1]
    l1 = wl1.shape[1]
    l2 = wl2.shape[1]
    l3 = wl3.shape[1]
    grid = (n // tm, n // tk)
    return pl.pallas_call(
        _conv2_mlp_kernel,
        out_shape=jax.ShapeDtypeStruct((n, l3), jnp.float32),
        grid=grid,
        in_specs=[
            pl.BlockSpec((tm, tk), lambda i, k: (i, k)),
            pl.BlockSpec((tk, f2), lambda i, k: (k, 0)),
            pl.BlockSpec((tm, f2), lambda i, k: (i, 0)),
            pl.BlockSpec((tm, f2), lambda i, k: (i, 0)),
            pl.BlockSpec((tm, 1), lambda i, k: (i, 0)),
            pl.BlockSpec((f2, l1), lambda i, k: (0, 0)),
            pl.BlockSpec((1, l1), lambda i, k: (0, 0)),
            pl.BlockSpec((l1, l2), lambda i, k: (0, 0)),
            pl.BlockSpec((1, l2), lambda i, k: (0, 0)),
            pl.BlockSpec((l2, l3), lambda i, k: (0, 0)),
            pl.BlockSpec((1, l3), lambda i, k: (0, 0)),
        ],
        out_specs=pl.BlockSpec((tm, l3), lambda i, k: (i, 0)),
        scratch_shapes=[pltpu.VMEM((tm, f2), jnp.float32)],
        compiler_params=_compiler_params(("parallel", "arbitrary")),
    )(a, q, q, t, dinv, wl1, bl1, wl2, bl2, wl3, bl3)


def kernel(x, edge_index, ws1, wa1, b1, ws2, wa2, b2,
           wl1, bl1, wl2, bl2, wl3, bl3):
    n = x.shape[0]
    tm, tk = 1024, 2048

    src, dst = edge_index[0], edge_index[1]
    not_self = (src != dst).astype(jnp.float32)
    a = jnp.zeros((n, n), jnp.float32).at[dst, src].add(not_self)
    deg = jnp.sum(a, axis=1, keepdims=True) + 1.0
    dinv = 1.0 / deg

    p, s = _proj(x, wa1, ws1, b1, tm=tm)
    q, t = _conv1(a, p, s, dinv, wa2, ws2, b2, tm=tm, tk=tk)
    out = _conv2_mlp(a, q, t, dinv, wl1, bl1, wl2, bl2, wl3, bl3,
                     tm=tm, tk=tk)
    return out[:, 0]
